# baseline (speedup 1.0000x reference)
import jax
import jax.numpy as jnp
from jax.experimental import pallas as pl
from jax.experimental.pallas import tpu as pltpu

_MEM = 1024
_TOPK = 4
_BB = 2048


def _ppo_block(x_ref, kT_ref, vmT_ref, W1T_ref, b1T_ref, WvT_ref, bv_ref,
               W2_ref, b2_ref, Wpi_ref, bpi_ref, out_ref):
    xb = x_ref[...]
    kT = kT_ref[...]
    vmT = vmT_ref[...]
    bb = xb.shape[0]

    sim = jnp.dot(xb, kT)

    hT = jnp.maximum(jnp.dot(W1T_ref[...], vmT) + b1T_ref[...], 0.0)
    pv_row = jnp.dot(WvT_ref[...], hT) + bv_ref[...]

    iota = jax.lax.broadcasted_iota(jnp.int32, (bb, _MEM), 1)
    simw = sim
    idxs = []
    pvs = []
    zero = jnp.float32(0.0)
    for j in range(_TOPK):
        aj = jnp.argmax(simw, axis=1).astype(jnp.int32)[:, None]
        hitb = iota == aj
        pvj = jnp.sum(jnp.where(hitb, pv_row, zero), axis=1,
                      keepdims=True)
        if j + 1 < _TOPK:
            simw = jnp.where(hitb, -jnp.inf, simw)
        idxs.append(aj)
        pvs.append(pvj)

    p0 = pvs[0]
    one = jnp.float32(1.0)
    jstar = (jnp.where(pvs[1] < p0, one, zero)
             + jnp.where(pvs[2] < p0, one, zero)
             + jnp.where(pvs[3] < p0, one, zero))
    istar = jnp.where(jstar == 0.0, idxs[0],
                      jnp.where(jstar == 1.0, idxs[1],
                                jnp.where(jstar == 2.0, idxs[2], idxs[3])))
    starb = iota == istar
    vcols = [jnp.sum(jnp.where(starb, vmT[d:d + 1, :], zero), axis=1,
                     keepdims=True) for d in range(vmT.shape[0])]

    xc = jnp.concatenate([xb] + vcols, axis=1)
    h2 = jnp.maximum(jnp.dot(xc, W2_ref[...]) + b2_ref[...], 0.0)
    logits = jnp.dot(h2, Wpi_ref[...]) + bpi_ref[...]
    m = jnp.max(logits, axis=1, keepdims=True)
    e = jnp.exp(logits - m)
    out_ref[...] = e / jnp.sum(e, axis=1, keepdims=True)


def kernel(x, keys_mem, values_mem, W1, b1, W2, b2, Wpi, bpi, Wv, bv):
    B, S = x.shape
    H = W1.shape[1]
    grid = (B // _BB,)
    rep = lambda i: (0, 0)
    out = pl.pallas_call(
        _ppo_block,
        grid=grid,
        in_specs=[
            pl.BlockSpec((_BB, S), lambda i: (i, 0)),
            pl.BlockSpec((S, _MEM), rep),
            pl.BlockSpec((S, _MEM), rep),
            pl.BlockSpec((H, S), rep),
            pl.BlockSpec((H, 1), rep),
            pl.BlockSpec((1, H), rep),
            pl.BlockSpec((1, 1), rep),
            pl.BlockSpec((2 * S, H), rep),
            pl.BlockSpec((1, H), rep),
            pl.BlockSpec((H, 2), rep),
            pl.BlockSpec((1, 2), rep),
        ],
        out_specs=pl.BlockSpec((_BB, 2), lambda i: (i, 0)),
        out_shape=jax.ShapeDtypeStruct((B, 2), jnp.float32),
        compiler_params=pltpu.CompilerParams(
            dimension_semantics=("parallel",)),
    )(x, keys_mem.T, values_mem.T, W1.T, b1.reshape(H, 1), Wv.T,
      bv.reshape(1, 1), W2, b2.reshape(1, H), Wpi, bpi.reshape(1, 2))
    return out


# speedup vs baseline: 1.0488x; 1.0488x over previous
import jax
import jax.numpy as jnp
from jax.experimental import pallas as pl
from jax.experimental.pallas import tpu as pltpu

_MEM = 1024
_TOPK = 4
_BB = 2048


def _ppo_block(x_ref, kT_ref, vmT_ref, W1T_ref, b1T_ref, WvT_ref, bv_ref,
               W2_ref, b2_ref, Wpi_ref, bpi_ref, out_ref):
    xb = x_ref[...]
    kT = kT_ref[...]
    vmT = vmT_ref[...]
    bb = xb.shape[0]

    sim = jnp.dot(xb, kT)

    hT = jnp.maximum(jnp.dot(W1T_ref[...], vmT) + b1T_ref[...], 0.0)
    pv_row = jnp.dot(WvT_ref[...], hT) + bv_ref[...]

    iota = jax.lax.broadcasted_iota(
        jnp.int32, (1, _MEM), 1).astype(jnp.float32)
    big = jnp.float32(2.0 * _MEM)
    simw = sim
    idxs = []
    pvs = []
    zero = jnp.float32(0.0)
    for j in range(_TOPK):
        mj = jnp.max(simw, axis=1, keepdims=True)
        aj = jnp.min(jnp.where(simw == mj, iota, big), axis=1,
                     keepdims=True)
        hitb = iota == aj
        pvj = jnp.sum(jnp.where(hitb, pv_row, zero), axis=1,
                      keepdims=True)
        if j + 1 < _TOPK:
            simw = jnp.where(hitb, -jnp.inf, simw)
        idxs.append(aj)
        pvs.append(pvj)

    p0 = pvs[0]
    one = jnp.float32(1.0)
    jstar = (jnp.where(pvs[1] < p0, one, zero)
             + jnp.where(pvs[2] < p0, one, zero)
             + jnp.where(pvs[3] < p0, one, zero))
    istar = jnp.where(jstar == 0.0, idxs[0],
                      jnp.where(jstar == 1.0, idxs[1],
                                jnp.where(jstar == 2.0, idxs[2], idxs[3])))
    starb = iota == istar
    vcols = [jnp.sum(jnp.where(starb, vmT[d:d + 1, :], zero), axis=1,
                     keepdims=True) for d in range(vmT.shape[0])]

    xc = jnp.concatenate([xb] + vcols, axis=1)
    h2 = jnp.maximum(jnp.dot(xc, W2_ref[...]) + b2_ref[...], 0.0)
    logits = jnp.dot(h2, Wpi_ref[...]) + bpi_ref[...]
    m = jnp.max(logits, axis=1, keepdims=True)
    e = jnp.exp(logits - m)
    out_ref[...] = e / jnp.sum(e, axis=1, keepdims=True)


def kernel(x, keys_mem, values_mem, W1, b1, W2, b2, Wpi, bpi, Wv, bv):
    B, S = x.shape
    H = W1.shape[1]
    grid = (B // _BB,)
    rep = lambda i: (0, 0)
    out = pl.pallas_call(
        _ppo_block,
        grid=grid,
        in_specs=[
            pl.BlockSpec((_BB, S), lambda i: (i, 0)),
            pl.BlockSpec((S, _MEM), rep),
            pl.BlockSpec((S, _MEM), rep),
            pl.BlockSpec((H, S), rep),
            pl.BlockSpec((H, 1), rep),
            pl.BlockSpec((1, H), rep),
            pl.BlockSpec((1, 1), rep),
            pl.BlockSpec((2 * S, H), rep),
            pl.BlockSpec((1, H), rep),
            pl.BlockSpec((H, 2), rep),
            pl.BlockSpec((1, 2), rep),
        ],
        out_specs=pl.BlockSpec((_BB, 2), lambda i: (i, 0)),
        out_shape=jax.ShapeDtypeStruct((B, 2), jnp.float32),
        compiler_params=pltpu.CompilerParams(
            dimension_semantics=("parallel",)),
    )(x, keys_mem.T, values_mem.T, W1.T, b1.reshape(H, 1), Wv.T,
      bv.reshape(1, 1), W2, b2.reshape(1, H), Wpi, bpi.reshape(1, 2))
    return out


# speedup vs baseline: 1.0656x; 1.0160x over previous
import jax
import jax.numpy as jnp
from jax.experimental import pallas as pl
from jax.experimental.pallas import tpu as pltpu

_MEM = 1024
_TOPK = 4
_BB = 4096


def _ppo_block(x_ref, kT_ref, vmT_ref, W1T_ref, b1T_ref, WvT_ref, bv_ref,
               W2_ref, b2_ref, Wpi_ref, bpi_ref, out_ref):
    xb = x_ref[...]
    kT = kT_ref[...]
    vmT = vmT_ref[...]
    bb = xb.shape[0]

    sim = jnp.dot(xb, kT)

    hT = jnp.maximum(jnp.dot(W1T_ref[...], vmT) + b1T_ref[...], 0.0)
    pv_row = jnp.dot(WvT_ref[...], hT) + bv_ref[...]

    iota = jax.lax.broadcasted_iota(
        jnp.int32, (1, _MEM), 1).astype(jnp.float32)
    big = jnp.float32(2.0 * _MEM)
    simw = sim
    idxs = []
    pvs = []
    zero = jnp.float32(0.0)
    for j in range(_TOPK):
        mj = jnp.max(simw, axis=1, keepdims=True)
        aj = jnp.min(jnp.where(simw == mj, iota, big), axis=1,
                     keepdims=True)
        hitb = iota == aj
        pvj = jnp.sum(jnp.where(hitb, pv_row, zero), axis=1,
                      keepdims=True)
        if j + 1 < _TOPK:
            simw = jnp.where(hitb, -jnp.inf, simw)
        idxs.append(aj)
        pvs.append(pvj)

    p0 = pvs[0]
    one = jnp.float32(1.0)
    jstar = (jnp.where(pvs[1] < p0, one, zero)
             + jnp.where(pvs[2] < p0, one, zero)
             + jnp.where(pvs[3] < p0, one, zero))
    istar = jnp.where(jstar == 0.0, idxs[0],
                      jnp.where(jstar == 1.0, idxs[1],
                                jnp.where(jstar == 2.0, idxs[2], idxs[3])))
    starb = iota == istar
    vcols = [jnp.sum(jnp.where(starb, vmT[d:d + 1, :], zero), axis=1,
                     keepdims=True) for d in range(vmT.shape[0])]

    xc = jnp.concatenate([xb] + vcols, axis=1)
    h2 = jnp.maximum(jnp.dot(xc, W2_ref[...]) + b2_ref[...], 0.0)
    logits = jnp.dot(h2, Wpi_ref[...]) + bpi_ref[...]
    m = jnp.max(logits, axis=1, keepdims=True)
    e = jnp.exp(logits - m)
    out_ref[...] = e / jnp.sum(e, axis=1, keepdims=True)


def kernel(x, keys_mem, values_mem, W1, b1, W2, b2, Wpi, bpi, Wv, bv):
    B, S = x.shape
    H = W1.shape[1]
    grid = (B // _BB,)
    rep = lambda i: (0, 0)
    out = pl.pallas_call(
        _ppo_block,
        grid=grid,
        in_specs=[
            pl.BlockSpec((_BB, S), lambda i: (i, 0)),
            pl.BlockSpec((S, _MEM), rep),
            pl.BlockSpec((S, _MEM), rep),
            pl.BlockSpec((H, S), rep),
            pl.BlockSpec((H, 1), rep),
            pl.BlockSpec((1, H), rep),
            pl.BlockSpec((1, 1), rep),
            pl.BlockSpec((2 * S, H), rep),
            pl.BlockSpec((1, H), rep),
            pl.BlockSpec((H, 2), rep),
            pl.BlockSpec((1, 2), rep),
        ],
        out_specs=pl.BlockSpec((_BB, 2), lambda i: (i, 0)),
        out_shape=jax.ShapeDtypeStruct((B, 2), jnp.float32),
        compiler_params=pltpu.CompilerParams(
            dimension_semantics=("parallel",)),
    )(x, keys_mem.T, values_mem.T, W1.T, b1.reshape(H, 1), Wv.T,
      bv.reshape(1, 1), W2, b2.reshape(1, H), Wpi, bpi.reshape(1, 2))
    return out
